# baseline (speedup 1.0000x reference)
import functools

import jax
import jax.numpy as jnp
from jax.experimental import pallas as pl
from jax.experimental.pallas import tpu as pltpu

N_ENT = 2048
N_REL = 8
D = 128
H = 128

BN = 512
BM = 512


def _mid_body(agg_ref, W_out_ref, relations_ref, W_relmap_ref, emb_ref, rel_emb_ref):
    a = agg_ref[...]
    e = jnp.where(a > 0, a, jnp.exp(jnp.minimum(a, 0.0)) - 1.0)
    emb_ref[...] = jax.lax.dot_general(
        e, W_out_ref[...], (((1,), (0,)), ((), ())),
        preferred_element_type=jnp.float32)
    rel_emb_ref[...] = jax.lax.dot_general(
        relations_ref[...], W_relmap_ref[...], (((1,), (0,)), ((), ())),
        preferred_element_type=jnp.float32)


def _mid(agg, W_out, relations, W_relmap):
    return pl.pallas_call(
        _mid_body,
        out_shape=(
            jax.ShapeDtypeStruct((N_ENT, D), jnp.float32),
            jax.ShapeDtypeStruct((N_REL, D), jnp.float32),
        ),
    )(agg, W_out, relations, W_relmap)


def _decoder_body(emb_n_ref, rel_ref, emb_m_ref, out_ref):
    r = pl.program_id(0)
    rel_row = rel_ref[pl.ds(r, 1), :]
    a = emb_n_ref[...] * rel_row
    out_ref[0] = jax.lax.dot_general(
        a, emb_m_ref[...], (((1,), (1,)), ((), ())),
        preferred_element_type=jnp.float32)


def _decoder(emb, rel_emb):
    grid = (N_REL, N_ENT // BN, N_ENT // BM)
    return pl.pallas_call(
        _decoder_body,
        grid=grid,
        in_specs=[
            pl.BlockSpec((BN, D), lambda r, i, j: (i, 0)),
            pl.BlockSpec((N_REL, D), lambda r, i, j: (0, 0)),
            pl.BlockSpec((BM, D), lambda r, i, j: (j, 0)),
        ],
        out_specs=pl.BlockSpec((1, BN, BM), lambda r, i, j: (r, i, j)),
        out_shape=jax.ShapeDtypeStruct((N_REL, N_ENT, N_ENT), jnp.float32),
    )(emb, rel_emb, emb)


def kernel(entities, relations, x_coo, W_src, W_dst, W_rel, att, W_out, W_relmap):
    x = x_coo.astype(jnp.int32)
    src, rel, dst = x[:, 0], x[:, 1], x[:, 2]
    h_src = entities @ W_src
    h_dst = entities @ W_dst
    r_e = (relations @ W_rel)[rel]
    z = jax.nn.leaky_relu(h_src[src] + h_dst[dst] + r_e, negative_slope=0.2)
    logits = z @ att
    m = jax.ops.segment_max(logits, dst, num_segments=N_ENT)
    m = jnp.where(jnp.isfinite(m), m, 0.0)
    ex = jnp.exp(logits - m[dst])
    denom = jax.ops.segment_sum(ex, dst, num_segments=N_ENT)
    alpha = ex / (denom[dst] + 1e-16)
    agg = jax.ops.segment_sum(alpha[:, None] * h_src[src], dst, num_segments=N_ENT)
    emb, rel_emb = _mid(agg, W_out, relations, W_relmap)
    return _decoder(emb, rel_emb)


# speedup vs baseline: 1.8674x; 1.8674x over previous
import jax
import jax.numpy as jnp
from jax import lax
from jax.experimental import pallas as pl
from jax.experimental.pallas import tpu as pltpu
from jax.experimental.pallas import tpu_sc as plsc

N_ENT = 2048
N_REL = 8
N_EDGE = 65536
D = 128
H = 128

NC = 2
NS = 16
NW = NC * NS
E_PER_W = N_EDGE // NW
CHUNK = 128
N_CHUNKS = E_PER_W // CHUNK
NSTRIP = 4
SW = D // NSTRIP
ACC_W = N_ENT * SW
DEN_W = N_ENT * 16

BN = 512
BM = 512

_SC_MESH = plsc.VectorSubcoreMesh(core_axis_name="c", subcore_axis_name="s")

_GDN = lax.GatherDimensionNumbers(
    offset_dims=(), collapsed_slice_dims=(0,), start_index_map=(0,))


def _vsplat(vec, lane):
    idx = jnp.full((16, 1), lane, jnp.int32)
    return lax.gather(vec, idx, _GDN, (1,),
                      mode=lax.GatherScatterMode.PROMISE_IN_BOUNDS)


def _passA_body(h_src, h_dst, rtab, att, srci, dsti, reli,
                accp_out,
                src_v, dst_v, rel_v, srows, drows, rrows, att_v, accbuf, sem):
    cid = lax.axis_index("c")
    sid = lax.axis_index("s")
    base = (cid * NS + sid) * E_PER_W

    pltpu.sync_copy(att, att_v)

    def _chunk(k, _):
        off = base + k * CHUNK
        pltpu.sync_copy(srci.at[pl.ds(off, CHUNK)], src_v)
        pltpu.sync_copy(dsti.at[pl.ds(off, CHUNK)], dst_v)
        pltpu.sync_copy(reli.at[pl.ds(off, CHUNK)], rel_v)
        pltpu.async_copy(h_src.at[src_v], srows, sem).wait()
        pltpu.async_copy(h_dst.at[dst_v], drows, sem).wait()
        pltpu.async_copy(rtab.at[rel_v], rrows, sem).wait()

        def _edge(e, _):
            acc = jnp.zeros((16,), jnp.float32)
            for j in range(H // 16):
                s = srows[e, pl.ds(j * 16, 16)]
                d = drows[e, pl.ds(j * 16, 16)]
                r = rrows[e, pl.ds(j * 16, 16)]
                x = s + d + r
                y = jnp.maximum(x, 0.2 * x)
                acc = acc + y * att_v[pl.ds(j * 16, 16)]
            accbuf[pl.ds(e * 16, 16)] = acc
            return 0

        lax.fori_loop(0, CHUNK, _edge, 0)
        pltpu.sync_copy(accbuf, accp_out.at[pl.ds(off * 16, CHUNK * 16)])
        return 0

    lax.fori_loop(0, N_CHUNKS, _chunk, 0)


def _passA(h_src, h_dst, rtab, att, srci, dsti, reli):
    f = pl.kernel(
        _passA_body,
        out_type=jax.ShapeDtypeStruct((N_EDGE * 16,), jnp.float32),
        mesh=_SC_MESH,
        scratch_types=[
            pltpu.VMEM((CHUNK,), jnp.int32),
            pltpu.VMEM((CHUNK,), jnp.int32),
            pltpu.VMEM((CHUNK,), jnp.int32),
            pltpu.VMEM((CHUNK, D), jnp.float32),
            pltpu.VMEM((CHUNK, D), jnp.float32),
            pltpu.VMEM((CHUNK, D), jnp.float32),
            pltpu.VMEM((H,), jnp.float32),
            pltpu.VMEM((CHUNK * 16,), jnp.float32),
            pltpu.SemaphoreType.DMA,
        ],
    )
    return f(h_src, h_dst, rtab, att, srci, dsti, reli)


def _red_body(accp_ref, exw_ref):
    a = accp_ref[...]
    rr = lax.broadcasted_iota(jnp.int32, (128, 128), 0) // 16
    cc = lax.broadcasted_iota(jnp.int32, (128, 128), 1) // 16
    blk = (rr == cc).astype(jnp.float32)
    lg = jax.lax.dot_general(
        a, blk, (((1,), (0,)), ((), ())),
        preferred_element_type=jnp.float32)
    gmax = jnp.max(lg)
    exw_ref[...] = jnp.exp(lg - gmax)


def _red(accp2d):
    return pl.pallas_call(
        _red_body,
        out_shape=jax.ShapeDtypeStruct((N_EDGE * 16 // 128, 128), jnp.float32),
    )(accp2d)


def _passB_body(h_src, srci, dsti, exwf,
                acc_out, den_out,
                src_v, dst_v, exw_v, srows, acc_v, den_v, sem):
    cid = lax.axis_index("c")
    sid = lax.axis_index("s")
    wid = cid * NS + sid
    base = wid * E_PER_W
    lanes = lax.iota(jnp.int32, 16)

    def _zden(i, _):
        den_v[pl.ds(i * 16, 16)] = jnp.zeros((16,), jnp.float32)
        return 0
    lax.fori_loop(0, DEN_W // 16, _zden, 0)

    for s in range(NSTRIP):
        def _zacc(i, _):
            acc_v[pl.ds(i * 16, 16)] = jnp.zeros((16,), jnp.float32)
            return 0
        lax.fori_loop(0, ACC_W // 16, _zacc, 0)

        def _chunk(k, _):
            off = base + k * CHUNK
            pltpu.sync_copy(srci.at[pl.ds(off, CHUNK)], src_v)
            pltpu.sync_copy(dsti.at[pl.ds(off, CHUNK)], dst_v)
            pltpu.sync_copy(exwf.at[pl.ds(off * 16, CHUNK * 16)], exw_v)
            pltpu.async_copy(h_src.at[src_v], srows, sem).wait()

            def _grp(g, _):
                dvec = dst_v[pl.ds(g * 16, 16)] * SW
                for e16 in range(16):
                    e = g * 16 + e16
                    sel = _vsplat(dvec, e16)
                    ex = exw_v[pl.ds(e * 16, 16)]
                    srow = srows.at[e]
                    for t in range(SW // 16):
                        sv = srow[pl.ds(s * SW + t * 16, 16)]
                        plsc.addupdate_scatter(
                            acc_v, [sel + (t * 16) + lanes], sv * ex)
                    if s == 0:
                        plsc.addupdate_scatter(
                            den_v, [jnp.right_shift(sel, 1) + lanes], ex)
                return 0

            lax.fori_loop(0, CHUNK // 16, _grp, 0)
            return 0

        lax.fori_loop(0, N_CHUNKS, _chunk, 0)
        pltpu.sync_copy(acc_v,
                        acc_out.at[pl.ds((s * NW + wid) * ACC_W, ACC_W)])

    pltpu.sync_copy(den_v, den_out.at[pl.ds(wid * DEN_W, DEN_W)])


def _passB(h_src, srci, dsti, exwf):
    f = pl.kernel(
        _passB_body,
        out_type=(
            jax.ShapeDtypeStruct((NSTRIP * NW * ACC_W,), jnp.float32),
            jax.ShapeDtypeStruct((NW * DEN_W,), jnp.float32),
        ),
        mesh=_SC_MESH,
        compiler_params=pltpu.CompilerParams(needs_layout_passes=False),
        scratch_types=[
            pltpu.VMEM((CHUNK,), jnp.int32),
            pltpu.VMEM((CHUNK,), jnp.int32),
            pltpu.VMEM((CHUNK * 16,), jnp.float32),
            pltpu.VMEM((CHUNK, D), jnp.float32),
            pltpu.VMEM((ACC_W,), jnp.float32),
            pltpu.VMEM((DEN_W,), jnp.float32),
            pltpu.SemaphoreType.DMA,
        ],
    )
    return f(h_src, srci, dsti, exwf)


def _pre_body(ent_ref, Ws_ref, Wd_ref, rel_ref, Wr_ref, hs_ref, hd_ref, rt_ref):
    e = ent_ref[...]
    hs_ref[...] = jax.lax.dot_general(
        e, Ws_ref[...], (((1,), (0,)), ((), ())),
        preferred_element_type=jnp.float32)
    hd_ref[...] = jax.lax.dot_general(
        e, Wd_ref[...], (((1,), (0,)), ((), ())),
        preferred_element_type=jnp.float32)
    rt_ref[...] = jax.lax.dot_general(
        rel_ref[...], Wr_ref[...], (((1,), (0,)), ((), ())),
        preferred_element_type=jnp.float32)


def _pre(entities, W_src, W_dst, relations, W_rel):
    return pl.pallas_call(
        _pre_body,
        out_shape=(
            jax.ShapeDtypeStruct((N_ENT, H), jnp.float32),
            jax.ShapeDtypeStruct((N_ENT, H), jnp.float32),
            jax.ShapeDtypeStruct((N_REL, H), jnp.float32),
        ),
    )(entities, W_src, W_dst, relations, W_rel)


def _merge_body(accs_ref, agg_ref):
    w = pl.program_id(1)

    @pl.when(w == 0)
    def _():
        agg_ref[...] = jnp.zeros_like(agg_ref)

    agg_ref[...] += accs_ref[0]


def _merge(accs):
    return pl.pallas_call(
        _merge_body,
        grid=(NSTRIP, NW),
        in_specs=[pl.BlockSpec((1, 1, N_ENT, SW), lambda s, w: (s, w, 0, 0))],
        out_specs=pl.BlockSpec((1, N_ENT, SW), lambda s, w: (s, 0, 0)),
        out_shape=jax.ShapeDtypeStruct((NSTRIP, N_ENT, SW), jnp.float32),
    )(accs)


def _dmerge_body(denp_ref, den_ref):
    w = pl.program_id(0)

    @pl.when(w == 0)
    def _():
        den_ref[...] = jnp.zeros_like(den_ref)

    den_ref[...] += denp_ref[0]


def _dmerge(denp):
    return pl.pallas_call(
        _dmerge_body,
        grid=(NW,),
        in_specs=[pl.BlockSpec((1, N_ENT, 16), lambda w: (w, 0, 0))],
        out_specs=pl.BlockSpec((N_ENT, 16), lambda w: (0, 0)),
        out_shape=jax.ShapeDtypeStruct((N_ENT, 16), jnp.float32),
    )(denp)


def _mid_body(aggs_ref, den_ref, W_out_ref, relations_ref, W_relmap_ref,
              emb_ref, rel_emb_ref):
    den = den_ref[...][:, 0:1]
    agg = aggs_ref[...] / (den + 1e-16)
    e = jnp.where(agg > 0, agg, jnp.exp(jnp.minimum(agg, 0.0)) - 1.0)
    emb_ref[...] = jax.lax.dot_general(
        e, W_out_ref[...], (((1,), (0,)), ((), ())),
        preferred_element_type=jnp.float32)
    rel_emb_ref[...] = jax.lax.dot_general(
        relations_ref[...], W_relmap_ref[...], (((1,), (0,)), ((), ())),
        preferred_element_type=jnp.float32)


def _mid(aggs, den, W_out, relations, W_relmap):
    return pl.pallas_call(
        _mid_body,
        out_shape=(
            jax.ShapeDtypeStruct((N_ENT, D), jnp.float32),
            jax.ShapeDtypeStruct((N_REL, D), jnp.float32),
        ),
    )(aggs, den, W_out, relations, W_relmap)


def _decoder_body(emb_n_ref, rel_ref, emb_m_ref, out_ref):
    r = pl.program_id(0)
    rel_row = rel_ref[pl.ds(r, 1), :]
    a = emb_n_ref[...] * rel_row
    out_ref[0] = jax.lax.dot_general(
        a, emb_m_ref[...], (((1,), (1,)), ((), ())),
        preferred_element_type=jnp.float32)


def _decoder(emb, rel_emb):
    grid = (N_REL, N_ENT // BN, N_ENT // BM)
    return pl.pallas_call(
        _decoder_body,
        grid=grid,
        in_specs=[
            pl.BlockSpec((BN, D), lambda r, i, j: (i, 0)),
            pl.BlockSpec((N_REL, D), lambda r, i, j: (0, 0)),
            pl.BlockSpec((BM, D), lambda r, i, j: (j, 0)),
        ],
        out_specs=pl.BlockSpec((1, BN, BM), lambda r, i, j: (r, i, j)),
        out_shape=jax.ShapeDtypeStruct((N_REL, N_ENT, N_ENT), jnp.float32),
    )(emb, rel_emb, emb)


def kernel(entities, relations, x_coo, W_src, W_dst, W_rel, att, W_out, W_relmap):
    x = x_coo.astype(jnp.int32)
    src = x[:, 0] + 0
    rel = x[:, 1] + 0
    dst = x[:, 2] + 0
    h_src, h_dst, rtab = _pre(entities, W_src, W_dst, relations, W_rel)
    accp = _passA(h_src, h_dst, rtab, att, src, dst, rel)
    exw = _red(accp.reshape(N_EDGE * 16 // 128, 128))
    accs, denp = _passB(h_src, src, dst, exw.reshape(-1))
    aggs = _merge(accs.reshape(NSTRIP, NW, N_ENT, SW))
    agg = aggs.transpose(1, 0, 2).reshape(N_ENT, D)
    den = _dmerge(denp.reshape(NW, N_ENT, 16))
    emb, rel_emb = _mid(agg, den, W_out, relations, W_relmap)
    return _decoder(emb, rel_emb)
